# Initial kernel scaffold; baseline (speedup 1.0000x reference)
import jax
import jax.numpy as jnp
from jax.experimental import pallas as pl


def kernel(x, W, b):
    raise NotImplementedError("write your pallas kernel here")


# speedup vs baseline: 687.6649x; 687.6649x over previous
import functools

import jax
import jax.numpy as jnp
from jax import lax
from jax.experimental import pallas as pl
from jax.experimental.pallas import tpu as pltpu
from jax.experimental.pallas import tpu_sc as plsc

K_NN = 4

SC_CORES = 2
SC_SUBCORES = 16
NW = SC_CORES * SC_SUBCORES

ROWS = 256
SUB = 128


def _tc_body(T, xr_ref, xb_ref, wt_ref, gidx_ref, yt_ref):
    b = pl.program_id(0)
    xr = xr_ref[0]
    xb = xb_ref[0]
    dot = lax.dot_general(xr, xb, (((0,), (0,)), ((), ())),
                          preferred_element_type=jnp.float32)
    nb = jnp.sum(xb * xb, axis=0, keepdims=True)
    nr = jnp.sum(xr * xr, axis=0)[:, None]
    dist = nr + nb - 2.0 * dot
    iota = lax.broadcasted_iota(jnp.int32, (ROWS, T), 1)
    base = b * T
    for k in range(K_NN):
        mv = jnp.min(dist, axis=1, keepdims=True)
        am = jnp.min(jnp.where(dist == mv, iota, T), axis=1)
        gidx_ref[0, k, :] = (base + am) * K_NN + k
        dist = jnp.where(iota == am[:, None], jnp.inf, dist)
    yt_ref[0] = lax.dot_general(xr, wt_ref[...], (((0,), (0,)), ((), ())),
                                preferred_element_type=jnp.float32)


def _tc_call(x, wt):
    B, C, T = x.shape
    KC = K_NN * C
    grid = (B, T // ROWS)
    return pl.pallas_call(
        functools.partial(_tc_body, T),
        grid=grid,
        in_specs=[
            pl.BlockSpec((1, C, ROWS), lambda b, j: (b, 0, j)),
            pl.BlockSpec((1, C, T), lambda b, j: (b, 0, 0)),
            pl.BlockSpec((C, KC), lambda b, j: (0, 0)),
        ],
        out_specs=[
            pl.BlockSpec((1, 8, ROWS), lambda b, j: (b, 0, j)),
            pl.BlockSpec((1, ROWS, KC), lambda b, j: (b, j, 0)),
        ],
        out_shape=[
            jax.ShapeDtypeStruct((B, 8, T), jnp.int32),
            jax.ShapeDtypeStruct((B, T, KC), jnp.float32),
        ],
    )(x, x, wt)


def _sc_body(T, n_tok, gidx_hbm, yflat_hbm, bias_hbm, out_hbm,
             idx_v, g_v, out_v, bias_v, sem):
    C = 128
    cid = lax.axis_index("c")
    sid = lax.axis_index("s")
    wid = sid * SC_CORES + cid
    tok_per_w = n_tok // NW
    quarters = T // tok_per_w
    b = wid // quarters
    t_base = (wid % quarters) * tok_per_w
    pltpu.sync_copy(bias_hbm, bias_v)
    for chunk in range(tok_per_w // SUB):
        t0 = t_base + chunk * SUB
        for k in range(K_NN):
            pltpu.sync_copy(gidx_hbm.at[b, k, pl.ds(t0, SUB)], idx_v.at[k])
        copies = [
            pltpu.async_copy(yflat_hbm.at[idx_v.at[k]], g_v.at[k], sem)
            for k in range(K_NN)
        ]
        for cp in copies:
            cp.wait()

        def body(t, carry):
            for o in range(C // 16):
                sl = pl.ds(o * 16, 16)
                acc = g_v[0, t, sl] + g_v[1, t, sl]
                acc = acc + g_v[2, t, sl]
                acc = acc + g_v[3, t, sl]
                out_v[t, sl] = acc + bias_v[sl]
            return carry

        lax.fori_loop(0, SUB, body, 0)
        pltpu.sync_copy(out_v, out_hbm.at[pl.ds(b * T + t0, SUB)])


def _sc_call(gidx, yflat, bias):
    C = yflat.shape[1]
    n_tok = yflat.shape[0] // K_NN
    T = gidx.shape[2]
    mesh = plsc.VectorSubcoreMesh(core_axis_name="c", subcore_axis_name="s")
    fn = functools.partial(
        pl.kernel,
        mesh=mesh,
        out_type=jax.ShapeDtypeStruct((n_tok, C), jnp.float32),
        scratch_types=[
            pltpu.VMEM((K_NN, SUB), jnp.int32),
            pltpu.VMEM((K_NN, SUB, C), jnp.float32),
            pltpu.VMEM((SUB, C), jnp.float32),
            pltpu.VMEM((C,), jnp.float32),
            pltpu.SemaphoreType.DMA,
        ],
    )(functools.partial(_sc_body, T, n_tok))
    return fn(gidx, yflat, bias)


def kernel(x, W, b):
    B, C, T = x.shape
    wt = W.transpose(1, 2, 0).reshape(C, K_NN * C)
    gidx, yt = _tc_call(x, wt)
    yflat = yt.reshape(B * T * K_NN, C)
    out_flat = _sc_call(gidx, yflat, b)
    return out_flat.reshape(B, T, C).transpose(0, 2, 1)
